# baseline (speedup 1.0000x reference)
import math

import jax
import jax.numpy as jnp
from jax.experimental import pallas as pl
from jax.experimental.pallas import tpu as pltpu

WINDOW = 15
S = 512
D = 256
NSPK = 9
T = 128
NT = S // T

_ACOS_C = (
    1.5707288 / math.pi,
    -0.2121144 / math.pi,
    0.0742610 / math.pi,
    -0.0187293 / math.pi,
)


def _wfun(cos):
    ax = jnp.abs(cos)
    p = jnp.float32(_ACOS_C[3])
    for c in _ACOS_C[2::-1]:
        p = p * ax + jnp.float32(c)
    r = jnp.sqrt(jnp.maximum(1.0 - ax, 0.0)) * p
    return jnp.where(cos >= 0.0, 1.0 - r, r)


def _adj_kernel(dia_ref, x_ref, q_ref, out_ref):
    b = pl.program_id(0)
    dl = dia_ref[b]
    xb = x_ref[0]
    xn = xb * jax.lax.rsqrt(
        jnp.maximum(jnp.sum(xb * xb, axis=1, keepdims=True), 1e-16)
    )

    q = q_ref[0]
    qmax = jnp.max(q, axis=0)
    io = jax.lax.broadcasted_iota(jnp.int32, (16, S), 0)
    spk = jnp.min(jnp.where(q >= qmax[None, :], io, 16), axis=0)
    oh = (io == spk[None, :]).astype(jnp.float32)

    ii0 = jax.lax.broadcasted_iota(jnp.int32, (T, T), 0)
    jj0 = jax.lax.broadcasted_iota(jnp.int32, (T, T), 1)
    band_mask = {
        ofs: (jnp.abs(ii0 - (jj0 + ofs)) <= WINDOW).astype(jnp.float32)
        for ofs in (-T, 0, T)
    }
    vcol = (jax.lax.broadcasted_iota(jnp.int32, (S, 1), 0) < dl).astype(
        jnp.float32
    )
    vrow = (jax.lax.broadcasted_iota(jnp.int32, (1, S), 1) < dl).astype(
        jnp.float32
    )

    dinv_parts = []
    prev_tiles = None
    for ti in range(NT):
        r0 = ti * T
        xr = xn[r0 : r0 + T]
        oh_r = oh[:, r0 : r0 + T]
        valid_r = vcol[r0 : r0 + T, :]
        tjs = [tj for tj in (ti - 1, ti, ti + 1) if 0 <= tj < NT]
        tiles = []
        spk_sum = None
        for tj in tjs:
            c0 = tj * T
            cos = jax.lax.dot_general(
                xr,
                xn[c0 : c0 + T],
                (((1,), (1,)), ((), ())),
                preferred_element_type=jnp.float32,
            )
            w = _wfun(cos)
            samef = jax.lax.dot_general(
                oh_r,
                oh[:, c0 : c0 + T],
                (((0,), (0,)), ((), ())),
                preferred_element_type=jnp.float32,
            )
            winf = band_mask[c0 - r0] * valid_r * vrow[:, c0 : c0 + T]
            spkf = winf * samef
            spk_sum = spkf if spk_sum is None else spk_sum + spkf
            tiles.append((c0, w, winf, spkf))
        cnt = jnp.sum(spk_sum, axis=1)
        gate = (cnt > 1.0).astype(jnp.float32)[:, None]
        pre_sum = None
        pres = []
        for c0, w, winf, spkf in tiles:
            pre = w * (winf + spkf * gate)
            pre_sum = pre if pre_sum is None else pre_sum + pre
            pres.append((c0, pre))
        deg = jnp.sum(pre_sum, axis=1)
        lo = tjs[0] * T
        hi = (tjs[-1] + 1) * T
        if lo > 0:
            out_ref[0, r0 : r0 + T, 0:lo] = jnp.zeros((T, lo), jnp.float32)
        if hi < S:
            out_ref[0, r0 : r0 + T, hi:S] = jnp.zeros((T, S - hi), jnp.float32)
        dinv_parts.append(jax.lax.rsqrt(jnp.where(deg == 0.0, 1.0, deg)))

        if prev_tiles is not None:
            p0 = (ti - 1) * T
            dr = dinv_parts[ti - 1][:, None]
            for c0, pre in prev_tiles:
                dc = dinv_parts[c0 // T][None, :]
                out_ref[0, p0 : p0 + T, c0 : c0 + T] = pre * dr * dc
        prev_tiles = pres

    p0 = (NT - 1) * T
    dr = dinv_parts[NT - 1][:, None]
    for c0, pre in prev_tiles:
        dc = dinv_parts[c0 // T][None, :]
        out_ref[0, p0 : p0 + T, c0 : c0 + T] = pre * dr * dc


def kernel(x, dia_len, qmask):
    B = x.shape[0]
    qt = jnp.transpose(qmask, (1, 2, 0))
    qt = jnp.concatenate(
        [qt, jnp.full((B, 16 - NSPK, S), -1.0, jnp.float32)], axis=1
    )
    dl = dia_len.astype(jnp.int32)
    grid_spec = pltpu.PrefetchScalarGridSpec(
        num_scalar_prefetch=1,
        grid=(B,),
        in_specs=[
            pl.BlockSpec((1, S, D), lambda b, d: (b, 0, 0)),
            pl.BlockSpec((1, 16, S), lambda b, d: (b, 0, 0)),
        ],
        out_specs=pl.BlockSpec((1, S, S), lambda b, d: (b, 0, 0)),
    )
    return pl.pallas_call(
        _adj_kernel,
        grid_spec=grid_spec,
        out_shape=jax.ShapeDtypeStruct((B, S, S), jnp.float32),
        compiler_params=pltpu.CompilerParams(
            dimension_semantics=("parallel",)
        ),
    )(dl, x, qt)


# speedup vs baseline: 1.0058x; 1.0058x over previous
import math

import jax
import jax.numpy as jnp
from jax.experimental import pallas as pl
from jax.experimental.pallas import tpu as pltpu

WINDOW = 15
S = 512
D = 256
NSPK = 9
T = 128
NT = S // T

_ACOS_C = (
    1.5707288 / math.pi,
    -0.2121144 / math.pi,
    0.0742610 / math.pi,
    -0.0187293 / math.pi,
)


def _wfun(cos):
    ax = jnp.abs(cos)
    p = jnp.float32(_ACOS_C[3])
    for c in _ACOS_C[2::-1]:
        p = p * ax + jnp.float32(c)
    r = jnp.sqrt(jnp.maximum(1.0 - ax, 0.0)) * p
    return jnp.where(cos >= 0.0, 1.0 - r, r)


def _adj_kernel(dia_ref, x_ref, q_ref, out_ref):
    b = pl.program_id(0)
    dl = dia_ref[b]
    xb = x_ref[0]
    xn = xb * jax.lax.rsqrt(
        jnp.maximum(jnp.sum(xb * xb, axis=1, keepdims=True), 1e-16)
    )

    q = q_ref[0]
    qmax = jnp.max(q, axis=0)
    io = jax.lax.broadcasted_iota(jnp.int32, (16, S), 0)
    spk = jnp.min(jnp.where(q >= qmax[None, :], io, 16), axis=0)

    def _bandm(rows, cols, ofs):
        ii = jax.lax.broadcasted_iota(jnp.int32, (rows, cols), 0) + ofs
        jj = jax.lax.broadcasted_iota(jnp.int32, (rows, cols), 1)
        return jnp.abs(ii - jj) <= WINDOW

    band_first = _bandm(T, 2 * T, 0)
    band_mid = _bandm(T, 3 * T, T)
    band_last = _bandm(T, 2 * T, T)
    vcol = jax.lax.broadcasted_iota(jnp.int32, (S, 1), 0) < dl
    vrow = jax.lax.broadcasted_iota(jnp.int32, (1, S), 1) < dl

    dinv_parts = []
    prev_slab = None
    for ti in range(NT):
        r0 = ti * T
        lo = max(ti - 1, 0) * T
        hi = min(ti + 2, NT) * T
        xr = xn[r0 : r0 + T]
        spk_r = spk[r0 : r0 + T]
        cos = jax.lax.dot_general(
            xr,
            xn[lo:hi],
            (((1,), (1,)), ((), ())),
            preferred_element_type=jnp.float32,
        )
        w = _wfun(cos)
        if ti == 0:
            band = band_first
        elif ti == NT - 1:
            band = band_last
        else:
            band = band_mid
        winm = band & vcol[r0 : r0 + T, :] & vrow[:, lo:hi]
        samet = spk_r[:, None] == spk[lo:hi][None, :]
        spkf = (winm & samet).astype(jnp.float32)
        winf = winm.astype(jnp.float32)
        cnt = jnp.sum(spkf, axis=1)
        gate = (cnt > 1.0).astype(jnp.float32)[:, None]
        pre = w * (winf + spkf * gate)
        deg = jnp.sum(pre, axis=1)
        if lo > 0:
            out_ref[0, r0 : r0 + T, 0:lo] = jnp.zeros((T, lo), jnp.float32)
        if hi < S:
            out_ref[0, r0 : r0 + T, hi:S] = jnp.zeros((T, S - hi), jnp.float32)
        dinv_parts.append(jax.lax.rsqrt(jnp.where(deg == 0.0, 1.0, deg)))

        if prev_slab is not None:
            p0, plo, ppre = prev_slab
            dr = dinv_parts[ti - 1][:, None]
            for k in range((ppre.shape[1]) // T):
                c0 = plo + k * T
                dc = dinv_parts[c0 // T][None, :]
                out_ref[0, p0 : p0 + T, c0 : c0 + T] = (
                    ppre[:, k * T : (k + 1) * T] * dr * dc
                )
        prev_slab = (r0, lo, pre)

    p0, plo, ppre = prev_slab
    dr = dinv_parts[NT - 1][:, None]
    for k in range((ppre.shape[1]) // T):
        c0 = plo + k * T
        dc = dinv_parts[c0 // T][None, :]
        out_ref[0, p0 : p0 + T, c0 : c0 + T] = (
            ppre[:, k * T : (k + 1) * T] * dr * dc
        )


def kernel(x, dia_len, qmask):
    B = x.shape[0]
    qt = jnp.transpose(qmask, (1, 2, 0))
    qt = jnp.concatenate(
        [qt, jnp.full((B, 16 - NSPK, S), -1.0, jnp.float32)], axis=1
    )
    dl = dia_len.astype(jnp.int32)
    grid_spec = pltpu.PrefetchScalarGridSpec(
        num_scalar_prefetch=1,
        grid=(B,),
        in_specs=[
            pl.BlockSpec((1, S, D), lambda b, d: (b, 0, 0)),
            pl.BlockSpec((1, 16, S), lambda b, d: (b, 0, 0)),
        ],
        out_specs=pl.BlockSpec((1, S, S), lambda b, d: (b, 0, 0)),
    )
    return pl.pallas_call(
        _adj_kernel,
        grid_spec=grid_spec,
        out_shape=jax.ShapeDtypeStruct((B, S, S), jnp.float32),
        compiler_params=pltpu.CompilerParams(
            dimension_semantics=("parallel",)
        ),
    )(dl, x, qt)
